# baseline (speedup 1.0000x reference)
import jax
import jax.numpy as jnp
from jax.experimental import pallas as pl
from jax.experimental.pallas import tpu as pltpu

_BINS = 32
_SIGMA = 0.05
_LOG2E = 1.4426950408889634
_C2 = -0.5 * _LOG2E / (_SIGMA * _SIGMA)

_CHUNK = 8192


def _hist_kernel(x_ref, bins_ref, o_ref):
    k = pl.program_id(1)
    bins_col = bins_ref[:, 0:1]
    xa = x_ref[0, 0:1, :]
    xb = x_ref[0, 1:2, :]
    da = xa - bins_col
    db = xb - bins_col
    ka = jnp.exp2(_C2 * da * da)
    kb = jnp.exp2(_C2 * db * db)
    h = jax.lax.dot_general(
        ka, kb, (((1,), (1,)), ((), ())),
        preferred_element_type=jnp.float32)

    @pl.when(k == 0)
    def _():
        o_ref[0] = h

    @pl.when(k != 0)
    def _():
        o_ref[0] = o_ref[0] + h


def kernel(image):
    n, c, h, w = image.shape
    hw = h * w
    chunk = min(_CHUNK, hw)
    num_k = hw // chunk
    x = image.reshape(n, c, hw)
    bins = jnp.broadcast_to(
        jnp.linspace(0.0, 1.0, _BINS, dtype=jnp.float32)[:, None],
        (_BINS, 128))
    out = pl.pallas_call(
        _hist_kernel,
        grid=(n, num_k),
        in_specs=[
            pl.BlockSpec((1, 2, chunk), lambda i, k: (i, 0, k)),
            pl.BlockSpec((_BINS, 128), lambda i, k: (0, 0)),
        ],
        out_specs=pl.BlockSpec((1, _BINS, _BINS), lambda i, k: (i, 0, 0)),
        out_shape=jax.ShapeDtypeStruct((n, _BINS, _BINS), jnp.float32),
        compiler_params=pltpu.CompilerParams(
            dimension_semantics=("parallel", "arbitrary")),
    )(x, bins)
    return out[:, None, :, :]


# speedup vs baseline: 1.7011x; 1.7011x over previous
import jax
import jax.numpy as jnp
from jax.experimental import pallas as pl
from jax.experimental.pallas import tpu as pltpu

_BINS = 32
_SIGMA = 0.05
_LOG2E = 1.4426950408889634
_C2 = -0.5 * _LOG2E / (_SIGMA * _SIGMA)

_CHUNK = 32768
_SUB = 2048


def _hist_kernel(x_ref, bins_ref, o_ref):
    k = pl.program_id(1)
    bins_col = bins_ref[:, 0:1]

    def sub_hist(s):
        xa = x_ref[0, 0:1, s * _SUB:(s + 1) * _SUB]
        xb = x_ref[0, 1:2, s * _SUB:(s + 1) * _SUB]
        da = xa - bins_col
        db = xb - bins_col
        ka = jnp.exp2(_C2 * da * da)
        kb = jnp.exp2(_C2 * db * db)
        return jax.lax.dot_general(
            ka, kb, (((1,), (1,)), ((), ())),
            preferred_element_type=jnp.float32)

    h = sub_hist(0)
    for s in range(1, _CHUNK // _SUB):
        h = h + sub_hist(s)

    @pl.when(k == 0)
    def _():
        o_ref[0] = h

    @pl.when(k != 0)
    def _():
        o_ref[0] = o_ref[0] + h


def kernel(image):
    n, c, h, w = image.shape
    hw = h * w
    chunk = min(_CHUNK, hw)
    assert chunk % _SUB == 0
    num_k = hw // chunk
    x = image.reshape(n, c, hw)
    bins = jnp.broadcast_to(
        jnp.linspace(0.0, 1.0, _BINS, dtype=jnp.float32)[:, None],
        (_BINS, 128))
    out = pl.pallas_call(
        _hist_kernel,
        grid=(n, num_k),
        in_specs=[
            pl.BlockSpec((1, 2, chunk), lambda i, k: (i, 0, k)),
            pl.BlockSpec((_BINS, 128), lambda i, k: (0, 0)),
        ],
        out_specs=pl.BlockSpec((1, _BINS, _BINS), lambda i, k: (i, 0, 0)),
        out_shape=jax.ShapeDtypeStruct((n, _BINS, _BINS), jnp.float32),
        compiler_params=pltpu.CompilerParams(
            dimension_semantics=("parallel", "arbitrary")),
    )(x, bins)
    return out[:, None, :, :]


# speedup vs baseline: 1.8194x; 1.0695x over previous
import jax
import jax.numpy as jnp
from jax.experimental import pallas as pl
from jax.experimental.pallas import tpu as pltpu

_BINS = 32
_SIGMA = 0.05
_LOG2E = 1.4426950408889634
_C2 = -0.5 * _LOG2E / (_SIGMA * _SIGMA)

_CHUNK = 65536
_SUB = 2048


def _hist_kernel(x_ref, bins_ref, o_ref):
    k = pl.program_id(1)
    bins_col = bins_ref[:, 0:1]

    def sub_hist(s):
        xa = x_ref[0, 0:1, s * _SUB:(s + 1) * _SUB]
        xb = x_ref[0, 1:2, s * _SUB:(s + 1) * _SUB]
        da = xa - bins_col
        db = xb - bins_col
        ka = jnp.exp2(_C2 * da * da)
        kb = jnp.exp2(_C2 * db * db)
        return jax.lax.dot_general(
            ka, kb, (((1,), (1,)), ((), ())),
            preferred_element_type=jnp.float32)

    h = sub_hist(0)
    for s in range(1, _CHUNK // _SUB):
        h = h + sub_hist(s)

    @pl.when(k == 0)
    def _():
        o_ref[0] = h

    @pl.when(k != 0)
    def _():
        o_ref[0] = o_ref[0] + h


def kernel(image):
    n, c, h, w = image.shape
    hw = h * w
    chunk = min(_CHUNK, hw)
    assert chunk % _SUB == 0
    num_k = hw // chunk
    x = image.reshape(n, c, hw)
    bins = jnp.broadcast_to(
        jnp.linspace(0.0, 1.0, _BINS, dtype=jnp.float32)[:, None],
        (_BINS, 128))
    out = pl.pallas_call(
        _hist_kernel,
        grid=(n, num_k),
        in_specs=[
            pl.BlockSpec((1, 2, chunk), lambda i, k: (i, 0, k)),
            pl.BlockSpec((_BINS, 128), lambda i, k: (0, 0)),
        ],
        out_specs=pl.BlockSpec((1, _BINS, _BINS), lambda i, k: (i, 0, 0)),
        out_shape=jax.ShapeDtypeStruct((n, _BINS, _BINS), jnp.float32),
        compiler_params=pltpu.CompilerParams(
            dimension_semantics=("parallel", "arbitrary")),
    )(x, bins)
    return out[:, None, :, :]


# speedup vs baseline: 2.6668x; 1.4658x over previous
import functools

import jax
import jax.numpy as jnp
from jax.experimental import pallas as pl
from jax.experimental.pallas import tpu as pltpu

_BINS = 32
_SIGMA = 0.05
_LOG2E = 1.4426950408889634
_C2 = -0.5 * _LOG2E / (_SIGMA * _SIGMA)

_BR = 128


def _hist_kernel(br, xa_ref, xb_ref, bins_ref, o_ref):
    k = pl.program_id(1)
    bins_col = bins_ref[:, 0:1]

    def row_hist(r):
        xa = xa_ref[0, r:r + 1, :]
        xb = xb_ref[0, r:r + 1, :]
        da = xa - bins_col
        db = xb - bins_col
        ka = jnp.exp2(_C2 * da * da)
        kb = jnp.exp2(_C2 * db * db)
        return jax.lax.dot_general(
            ka, kb, (((1,), (1,)), ((), ())),
            preferred_element_type=jnp.float32)

    h = row_hist(0)
    for r in range(1, br):
        h = h + row_hist(r)

    @pl.when(k == 0)
    def _():
        o_ref[0] = h

    @pl.when(k != 0)
    def _():
        o_ref[0] = o_ref[0] + h


def kernel(image):
    n, c, h, w = image.shape
    x = image.reshape(n * c, h, w)
    bins = jnp.broadcast_to(
        jnp.linspace(0.0, 1.0, _BINS, dtype=jnp.float32)[:, None],
        (_BINS, 128))
    br = min(_BR, h)
    num_k = h // br
    out = pl.pallas_call(
        functools.partial(_hist_kernel, br),
        grid=(n, num_k),
        in_specs=[
            pl.BlockSpec((1, br, w), lambda i, k: (2 * i, k, 0)),
            pl.BlockSpec((1, br, w), lambda i, k: (2 * i + 1, k, 0)),
            pl.BlockSpec((_BINS, 128), lambda i, k: (0, 0)),
        ],
        out_specs=pl.BlockSpec((1, _BINS, _BINS), lambda i, k: (i, 0, 0)),
        out_shape=jax.ShapeDtypeStruct((n, _BINS, _BINS), jnp.float32),
        compiler_params=pltpu.CompilerParams(
            dimension_semantics=("parallel", "arbitrary")),
    )(x, x, bins)
    return out[:, None, :, :]


# speedup vs baseline: 2.7098x; 1.0161x over previous
import functools

import jax
import jax.numpy as jnp
from jax.experimental import pallas as pl
from jax.experimental.pallas import tpu as pltpu

_BINS = 32
_SIGMA = 0.05
_LOG2E = 1.4426950408889634
_C2 = -0.5 * _LOG2E / (_SIGMA * _SIGMA)
_S = (0.5 * _LOG2E) ** 0.5 / _SIGMA

_BR = 128


def _hist_kernel(br, xa_ref, xb_ref, bins_ref, o_ref):
    k = pl.program_id(1)
    bins_col = bins_ref[:, 0:1]

    def row_hist(r):
        xa = xa_ref[0, r:r + 1, :] * _S
        xb = xb_ref[0, r:r + 1, :] * _S
        da = (xa - bins_col).astype(jnp.bfloat16)
        db = (xb - bins_col).astype(jnp.bfloat16)
        ka = jnp.exp2(-(da * da))
        kb = jnp.exp2(-(db * db))
        return jax.lax.dot_general(
            ka, kb, (((1,), (1,)), ((), ())),
            preferred_element_type=jnp.float32)

    h = row_hist(0)
    for r in range(1, br):
        h = h + row_hist(r)

    @pl.when(k == 0)
    def _():
        o_ref[0] = h

    @pl.when(k != 0)
    def _():
        o_ref[0] = o_ref[0] + h


def kernel(image):
    n, c, h, w = image.shape
    x = image.reshape(n * c, h, w)
    bins = jnp.broadcast_to(
        (jnp.linspace(0.0, 1.0, _BINS, dtype=jnp.float32) * _S)[:, None],
        (_BINS, 128))
    br = min(_BR, h)
    num_k = h // br
    out = pl.pallas_call(
        functools.partial(_hist_kernel, br),
        grid=(n, num_k),
        in_specs=[
            pl.BlockSpec((1, br, w), lambda i, k: (2 * i, k, 0)),
            pl.BlockSpec((1, br, w), lambda i, k: (2 * i + 1, k, 0)),
            pl.BlockSpec((_BINS, 128), lambda i, k: (0, 0)),
        ],
        out_specs=pl.BlockSpec((1, _BINS, _BINS), lambda i, k: (i, 0, 0)),
        out_shape=jax.ShapeDtypeStruct((n, _BINS, _BINS), jnp.float32),
        compiler_params=pltpu.CompilerParams(
            dimension_semantics=("parallel", "arbitrary")),
    )(x, x, bins)
    return out[:, None, :, :]


# speedup vs baseline: 2.8568x; 1.0542x over previous
import functools

import jax
import jax.numpy as jnp
from jax.experimental import pallas as pl
from jax.experimental.pallas import tpu as pltpu

_BINS = 32
_SIGMA = 0.05
_LOG2E = 1.4426950408889634
_C2 = -0.5 * _LOG2E / (_SIGMA * _SIGMA)
_S = (0.5 * _LOG2E) ** 0.5 / _SIGMA

_BR = 256


def _hist_kernel(br, xa_ref, xb_ref, bins_ref, o_ref):
    k = pl.program_id(1)
    bins_col = bins_ref[:, 0:1]

    def row_hist(r):
        xa = xa_ref[0, r:r + 1, :] * _S
        xb = xb_ref[0, r:r + 1, :] * _S
        da = (xa - bins_col).astype(jnp.bfloat16)
        db = (xb - bins_col).astype(jnp.bfloat16)
        ka = jnp.exp2(-(da * da))
        kb = jnp.exp2(-(db * db))
        return jax.lax.dot_general(
            ka, kb, (((1,), (1,)), ((), ())),
            preferred_element_type=jnp.float32)

    h = row_hist(0)
    for r in range(1, br):
        h = h + row_hist(r)

    @pl.when(k == 0)
    def _():
        o_ref[0] = h

    @pl.when(k != 0)
    def _():
        o_ref[0] = o_ref[0] + h


def kernel(image):
    n, c, h, w = image.shape
    x = image.reshape(n * c, h, w)
    bins = jnp.broadcast_to(
        (jnp.linspace(0.0, 1.0, _BINS, dtype=jnp.float32) * _S)[:, None],
        (_BINS, 128))
    br = min(_BR, h)
    num_k = h // br
    out = pl.pallas_call(
        functools.partial(_hist_kernel, br),
        grid=(n, num_k),
        in_specs=[
            pl.BlockSpec((1, br, w), lambda i, k: (2 * i, k, 0)),
            pl.BlockSpec((1, br, w), lambda i, k: (2 * i + 1, k, 0)),
            pl.BlockSpec((_BINS, 128), lambda i, k: (0, 0)),
        ],
        out_specs=pl.BlockSpec((1, _BINS, _BINS), lambda i, k: (i, 0, 0)),
        out_shape=jax.ShapeDtypeStruct((n, _BINS, _BINS), jnp.float32),
        compiler_params=pltpu.CompilerParams(
            dimension_semantics=("parallel", "arbitrary")),
    )(x, x, bins)
    return out[:, None, :, :]


# speedup vs baseline: 2.9379x; 1.0284x over previous
import functools

import jax
import jax.numpy as jnp
from jax.experimental import pallas as pl
from jax.experimental.pallas import tpu as pltpu

_BINS = 32
_SIGMA = 0.05
_LOG2E = 1.4426950408889634
_C2 = -0.5 * _LOG2E / (_SIGMA * _SIGMA)
_S = (0.5 * _LOG2E) ** 0.5 / _SIGMA

_BR = 512


def _hist_kernel(br, xa_ref, xb_ref, bins_ref, o_ref):
    k = pl.program_id(1)
    bins_col = bins_ref[:, 0:1]

    def row_hist(r):
        xa = xa_ref[0, r:r + 1, :] * _S
        xb = xb_ref[0, r:r + 1, :] * _S
        da = (xa - bins_col).astype(jnp.bfloat16)
        db = (xb - bins_col).astype(jnp.bfloat16)
        ka = jnp.exp2(-(da * da))
        kb = jnp.exp2(-(db * db))
        return jax.lax.dot_general(
            ka, kb, (((1,), (1,)), ((), ())),
            preferred_element_type=jnp.float32)

    h = row_hist(0)
    for r in range(1, br):
        h = h + row_hist(r)

    @pl.when(k == 0)
    def _():
        o_ref[0] = h

    @pl.when(k != 0)
    def _():
        o_ref[0] = o_ref[0] + h


def kernel(image):
    n, c, h, w = image.shape
    x = image.reshape(n * c, h, w)
    bins = jnp.broadcast_to(
        (jnp.linspace(0.0, 1.0, _BINS, dtype=jnp.float32) * _S)[:, None],
        (_BINS, 128))
    br = min(_BR, h)
    num_k = h // br
    out = pl.pallas_call(
        functools.partial(_hist_kernel, br),
        grid=(n, num_k),
        in_specs=[
            pl.BlockSpec((1, br, w), lambda i, k: (2 * i, k, 0)),
            pl.BlockSpec((1, br, w), lambda i, k: (2 * i + 1, k, 0)),
            pl.BlockSpec((_BINS, 128), lambda i, k: (0, 0)),
        ],
        out_specs=pl.BlockSpec((1, _BINS, _BINS), lambda i, k: (i, 0, 0)),
        out_shape=jax.ShapeDtypeStruct((n, _BINS, _BINS), jnp.float32),
        compiler_params=pltpu.CompilerParams(
            dimension_semantics=("parallel", "arbitrary")),
    )(x, x, bins)
    return out[:, None, :, :]
